# baseline (speedup 1.0000x reference)
import functools
import jax
import jax.numpy as jnp
import numpy as np
from jax.experimental import pallas as pl
from jax.experimental.pallas import tpu as pltpu
from jax.experimental.pallas import tpu_sc as plsc

B, T, D, H, DH, DFF = 2, 4096, 1024, 16, 64, 4096
R, C = 2, 64
NB = T // C
NC = T // C
G = 4
GR = G * C
NG = NC // G
RBH = R * B * H
BH = B * H


def _proj_body(x_ref, wc_ref, qkv_ref):
    qkv_ref[...] = jnp.dot(x_ref[...], wc_ref[...],
                           preferred_element_type=jnp.float32)


def _proj(x2, W_qk, W_v, interpret=False):
    Wc = jnp.concatenate([W_qk.reshape(D, H, 1, DH), W_v.reshape(D, H, 1, DH)],
                         axis=2).reshape(D, 2 * D)
    TM = 512
    grid = (B * T // TM,)
    return pl.pallas_call(
        _proj_body,
        grid=grid,
        in_specs=[
            pl.BlockSpec((TM, D), lambda i: (i, 0)),
            pl.BlockSpec((D, 2 * D), lambda i: (0, 0)),
        ],
        out_specs=pl.BlockSpec((TM, 2 * D), lambda i: (i, 0)),
        out_shape=jax.ShapeDtypeStruct((B * T, 2 * D), jnp.float32),
        interpret=interpret,
    )(x2.reshape(B * T, D), Wc)


def _bucket_body(qkr_ref, rot_ref, out_ref):
    rows = qkr_ref[:, :DH]
    rot = rot_ref[...]
    ro = jnp.dot(rows, rot, preferred_element_type=jnp.float32)
    cols = []
    half = NB // 2
    for r in range(R):
        rr = ro[:, r * half:(r + 1) * half]
        mp = jnp.max(rr, axis=-1)
        mn = jnp.max(-rr, axis=-1)
        ap = jnp.argmax(rr, axis=-1).astype(jnp.int32)
        an = jnp.argmax(-rr, axis=-1).astype(jnp.int32)
        b = jnp.where(mp >= mn, ap, half + an)
        cols.append(b[:, None])
    out_ref[...] = jnp.concatenate(cols, axis=1)


def _buckets(qkv, rotations, interpret=False):
    rot = rotations.transpose(1, 0, 2).reshape(DH, R * (NB // 2))
    TMH = 4096
    grid = (B * T * H // TMH,)
    out = pl.pallas_call(
        _bucket_body,
        grid=grid,
        in_specs=[
            pl.BlockSpec((TMH, 2 * DH), lambda i: (i, 0)),
            pl.BlockSpec((DH, R * (NB // 2)), lambda i: (0, 0)),
        ],
        out_specs=pl.BlockSpec((TMH, R), lambda i: (i, 0)),
        out_shape=jax.ShapeDtypeStruct((B * T * H, R), jnp.int32),
        interpret=interpret,
    )(qkv.reshape(B * T * H, 2 * DH), rot)
    return out.reshape(B, T, H, R).transpose(3, 0, 2, 1)


CH = 128
NCH = T // CH


def _sc_route_body(bkt_hbm, qkvrows_hbm,
                   sqkv_hbm, cb_hbm, dst_hbm,
                   bkt_v, rank_v, hist, off, ord_v, cb_v, dst_v, idx_v,
                   rowbuf, rowbuf2, sem, gsem, gsem2):
    ncores = 2
    wid = jax.lax.axis_index("s") * ncores + jax.lax.axis_index("c")
    lane = jax.lax.broadcasted_iota(jnp.int32, (16,), 0)
    bh = wid
    b = bh // H
    h = bh % H
    row_base = b * (T * H) + h
    pltpu.sync_copy(bkt_hbm.at[bh], bkt_v)
    for j in range(NB // 16):
        hist[pl.ds(j * 16, 16)] = jnp.zeros((16,), jnp.int32)

    def hist_step(i, _):
        b16 = bkt_v[pl.ds(i * 16, 16)]
        cnt, last = plsc.scan_count(b16)
        old = plsc.load_gather(hist, [b16])
        rank_v[pl.ds(i * 16, 16)] = old + cnt - 1
        plsc.addupdate_scatter(hist, [b16], cnt, mask=last)
        return 0

    jax.lax.fori_loop(0, T // 16, hist_step, 0)
    carry = jnp.int32(0)
    for j in range(NB // 16):
        h16 = hist[pl.ds(j * 16, 16)]
        inc = plsc.cumsum(h16) + carry
        off[pl.ds(j * 16, 16)] = inc - h16
        carry = carry + jnp.sum(h16)

    def perm_step(i, _):
        b16 = bkt_v[pl.ds(i * 16, 16)]
        r16 = rank_v[pl.ds(i * 16, 16)]
        d16 = plsc.load_gather(off, [b16]) + r16
        src = i * 16 + lane
        plsc.store_scatter(ord_v, [d16], src)
        plsc.store_scatter(cb_v, [d16], b16)
        dst_v[pl.ds(i * 16, 16)] = d16
        return 0

    jax.lax.fori_loop(0, T // 16, perm_step, 0)

    def idx_step(i, _):
        o16 = ord_v[pl.ds(i * 16, 16)]
        idx_v[pl.ds(i * 16, 16)] = o16 * H + row_base
        return 0

    jax.lax.fori_loop(0, T // 16, idx_step, 0)
    pltpu.sync_copy(cb_v, cb_hbm.at[bh])
    pltpu.sync_copy(dst_v, dst_hbm.at[bh])
    bufs = (rowbuf, rowbuf2)
    sems = (gsem, gsem2)
    cp = [None] * NCH
    for c in range(NCH):
        sl = pl.ds(c * CH, CH)
        cp[c] = pltpu.async_copy(qkvrows_hbm.at[idx_v.at[sl]],
                                 bufs[c % 2], sems[c % 2])
        if c >= 1:
            cp[c - 1].wait()
            pltpu.sync_copy(bufs[(c - 1) % 2],
                            sqkv_hbm.at[bh, pl.ds((c - 1) * CH, CH), :])
    cp[NCH - 1].wait()
    pltpu.sync_copy(bufs[(NCH - 1) % 2],
                    sqkv_hbm.at[bh, pl.ds((NCH - 1) * CH, CH), :])


def _sc_route(buckets_r, qkv):
    mesh = plsc.VectorSubcoreMesh(core_axis_name="c", subcore_axis_name="s")
    f = pl.kernel(
        _sc_route_body,
        compiler_params=pltpu.CompilerParams(needs_layout_passes=False),
        out_type=[
            jax.ShapeDtypeStruct((BH, T, 2 * DH), jnp.float32),
            jax.ShapeDtypeStruct((BH, T), jnp.int32),
            jax.ShapeDtypeStruct((BH, T), jnp.int32),
        ],
        mesh=mesh,
        scratch_types=[
            pltpu.VMEM((T,), jnp.int32),
            pltpu.VMEM((T,), jnp.int32),
            pltpu.VMEM((NB,), jnp.int32),
            pltpu.VMEM((NB,), jnp.int32),
            pltpu.VMEM((T,), jnp.int32),
            pltpu.VMEM((T,), jnp.int32),
            pltpu.VMEM((T,), jnp.int32),
            pltpu.VMEM((T,), jnp.int32),
            pltpu.VMEM((CH, 2 * DH), jnp.float32),
            pltpu.VMEM((CH, 2 * DH), jnp.float32),
            pltpu.SemaphoreType.DMA,
            pltpu.SemaphoreType.DMA,
            pltpu.SemaphoreType.DMA,
        ],
    )
    return f(buckets_r.reshape(BH, T), qkv.reshape(B * T * H, 2 * DH))


def _attn_body(sqkv_ref, bl_ref, o_ref, l_ref):
    scale = 1.0 / np.sqrt(DH)
    ione = jnp.ones((1, 1), jnp.float32)
    dn0 = (((0,), (0,)), ((), ()))
    qc = jax.lax.broadcasted_iota(jnp.int32, (GR, 1), 0) // C
    kcB = jax.lax.broadcasted_iota(jnp.int32, (1, GR), 1) // C
    allowB = (kcB == qc) | (kcB == qc - 1)
    allowP = qc == 0
    eye = (jax.lax.broadcasted_iota(jnp.int32, (GR, 1), 0)
           == jax.lax.broadcasted_iota(jnp.int32, (1, GR), 1))
    for g in range(NG):
        base = g * GR
        wstart = (base - C) % T
        qs = sqkv_ref[0, pl.ds(base, GR), :DH]
        kp = sqkv_ref[0, pl.ds(wstart, C), :DH]
        vb = sqkv_ref[0, pl.ds(base, GR), DH:]
        vp = sqkv_ref[0, pl.ds(wstart, C), DH:]
        bkb = bl_ref[0, :, pl.ds(base, GR)]
        bkp = bl_ref[0, :, pl.ds(wstart, C)]
        bq = jax.lax.dot_general(bkb.astype(jnp.float32), ione, dn0,
                                 preferred_element_type=jnp.float32)
        bq = bq.astype(jnp.int32)

        knb = qs / (jnp.sqrt(jnp.sum(qs * qs, axis=-1, keepdims=True)) + 1e-6)
        knp = kp / (jnp.sqrt(jnp.sum(kp * kp, axis=-1, keepdims=True)) + 1e-6)

        dn = (((1,), (1,)), ((), ()))
        qsb = qs.astype(jnp.bfloat16)
        dotsB = jax.lax.dot_general(qsb, knb.astype(jnp.bfloat16), dn,
                                    preferred_element_type=jnp.float32) * scale
        dotsP = jax.lax.dot_general(qsb, knp.astype(jnp.bfloat16), dn,
                                    preferred_element_type=jnp.float32) * scale

        maskB = (bq == bkb) & allowB
        dotsB = jnp.where(maskB, dotsB, -1e9)
        dotsB = jnp.where(eye, dotsB - 1e5, dotsB)

        maskP = (bq == bkp) & allowP
        dotsP = jnp.where(maskP, dotsP, -1e9)

        mB = jnp.max(dotsB, axis=-1, keepdims=True)
        mP = jnp.max(dotsP, axis=-1, keepdims=True)
        m = jnp.maximum(mB, mP)
        eB = jnp.exp(dotsB - m)
        eP = jnp.exp(dotsP - m)
        s = (jnp.sum(eB, axis=-1, keepdims=True)
             + jnp.sum(eP, axis=-1, keepdims=True))
        logits = m + jnp.log(s)
        o = (jnp.dot(eB.astype(jnp.bfloat16), vb.astype(jnp.bfloat16),
                     preferred_element_type=jnp.float32)
             + jnp.dot(eP.astype(jnp.bfloat16), vp.astype(jnp.bfloat16),
                       preferred_element_type=jnp.float32)) / s
        o_ref[0, pl.ds(base, GR), :DH] = o
        l_ref[0, pl.ds(base, GR), :] = logits


def _attention(sqkv, cb, interpret=False):
    grid = (BH,)
    o, l = pl.pallas_call(
        _attn_body,
        grid=grid,
        in_specs=[
            pl.BlockSpec((1, T, 2 * DH), lambda i: (i, 0, 0)),
            pl.BlockSpec((1, 1, T), lambda i: (i, 0, 0)),
        ],
        out_specs=[
            pl.BlockSpec((1, T, 2 * DH), lambda i: (i, 0, 0)),
            pl.BlockSpec((1, T, 1), lambda i: (i, 0, 0)),
        ],
        out_shape=[
            jax.ShapeDtypeStruct((BH, T, 2 * DH), jnp.float32),
            jax.ShapeDtypeStruct((BH, T, 1), jnp.float32),
        ],
        interpret=interpret,
    )(sqkv, cb.reshape(BH, 1, T))
    return o, l


def _sc_unsort_body(orows_hbm, l_hbm, dst_hbm,
                    ou_hbm, lu_hbm,
                    dst_v, idx_v, l_v, lu_v, rowbuf, rowbuf2,
                    sem, gsem, gsem2):
    ncores = 2
    wid = jax.lax.axis_index("s") * ncores + jax.lax.axis_index("c")
    bh = wid
    b = bh // H
    h = bh % H
    pltpu.sync_copy(dst_hbm.at[bh], dst_v)
    pltpu.sync_copy(l_hbm.at[bh], l_v)

    def idx_step(i, _):
        d16 = dst_v[pl.ds(i * 16, 16)]
        idx_v[pl.ds(i * 16, 16)] = d16 + bh * T
        lu_v[pl.ds(i * 16, 16)] = plsc.load_gather(l_v, [d16])
        return 0

    jax.lax.fori_loop(0, T // 16, idx_step, 0)
    pltpu.sync_copy(lu_v, lu_hbm.at[b, h, :])
    bufs = (rowbuf, rowbuf2)
    sems = (gsem, gsem2)
    cp = [None] * NCH
    for c in range(NCH):
        sl = pl.ds(c * CH, CH)
        cp[c] = pltpu.async_copy(orows_hbm.at[idx_v.at[sl]],
                                 bufs[c % 2], sems[c % 2])
        if c >= 1:
            cp[c - 1].wait()
            pltpu.sync_copy(bufs[(c - 1) % 2],
                            ou_hbm.at[b, h, pl.ds((c - 1) * CH, CH), :])
    cp[NCH - 1].wait()
    pltpu.sync_copy(bufs[(NCH - 1) % 2],
                    ou_hbm.at[b, h, pl.ds((NCH - 1) * CH, CH), :])


def _sc_unsort(o, l, dst):
    mesh = plsc.VectorSubcoreMesh(core_axis_name="c", subcore_axis_name="s")
    f = pl.kernel(
        _sc_unsort_body,
        compiler_params=pltpu.CompilerParams(needs_layout_passes=False),
        out_type=[
            jax.ShapeDtypeStruct((B, H, T, 2 * DH), jnp.float32),
            jax.ShapeDtypeStruct((B, H, T), jnp.float32),
        ],
        mesh=mesh,
        scratch_types=[
            pltpu.VMEM((T,), jnp.int32),
            pltpu.VMEM((T,), jnp.int32),
            pltpu.VMEM((T,), jnp.float32),
            pltpu.VMEM((T,), jnp.float32),
            pltpu.VMEM((CH, 2 * DH), jnp.float32),
            pltpu.VMEM((CH, 2 * DH), jnp.float32),
            pltpu.SemaphoreType.DMA,
            pltpu.SemaphoreType.DMA,
            pltpu.SemaphoreType.DMA,
        ],
    )
    o_u, l_u = f(o.reshape(BH * T, 2 * DH), l.reshape(BH, T), dst)
    return o_u, l_u


def _tail_body(x1_ref, x2_ref, o0_ref, o1_ref, l0_ref, l1_ref,
               wo_ref, w1_ref, b1_ref, w2_ref, b2_ref, y1_ref, y2_ref):
    l0 = l0_ref[0, :, :]
    l1 = l1_ref[0, :, :]
    m = jnp.maximum(l0, l1)
    e0 = jnp.exp(l0 - m)
    e1 = jnp.exp(l1 - m)
    den = e0 + e1
    w0 = e0 / den
    w1 = e1 / den
    ex = (jax.lax.broadcasted_iota(jnp.int32, (H, D), 1) // DH
          == jax.lax.broadcasted_iota(jnp.int32, (H, D), 0)).astype(jnp.float32)
    dn = (((0,), (0,)), ((), ()))
    w0e = jax.lax.dot_general(w0, ex, dn,
                              preferred_element_type=jnp.float32)
    w1e = jax.lax.dot_general(w1, ex, dn,
                              preferred_element_type=jnp.float32)
    parts = []
    for h in range(H):
        sl = slice(h * DH, (h + 1) * DH)
        parts.append(w0e[:, sl] * o0_ref[0, h, :, :DH]
                     + w1e[:, sl] * o1_ref[0, h, :, :DH])
    attn = jnp.concatenate(parts, axis=1)
    y1 = x1_ref[...] + jnp.dot(attn.astype(jnp.bfloat16), wo_ref[...],
                               preferred_element_type=jnp.float32)
    y1_ref[...] = y1
    hpre = jnp.dot(y1.astype(jnp.bfloat16), w1_ref[...],
                   preferred_element_type=jnp.float32)
    hpre = hpre + b1_ref[...]
    hh = jax.nn.gelu(hpre)
    y2 = x2_ref[...] + jnp.dot(hh.astype(jnp.bfloat16), w2_ref[...],
                               preferred_element_type=jnp.float32) + b2_ref[...]
    y2_ref[...] = y2


def _tail(x1, x2, o0, o1, l0, l1, W_o, W_ff1, b_ff1, W_ff2, b_ff2,
          interpret=False):
    TM = 256
    NT = T // TM
    grid = (B * T // TM,)
    ospec = pl.BlockSpec((1, H, TM, 2 * DH), lambda i: (i // NT, 0, i % NT, 0))
    lspec = pl.BlockSpec((1, H, TM), lambda i: (i // NT, 0, i % NT))
    y1, y2 = pl.pallas_call(
        _tail_body,
        grid=grid,
        in_specs=[
            pl.BlockSpec((TM, D), lambda i: (i, 0)),
            pl.BlockSpec((TM, D), lambda i: (i, 0)),
            ospec, ospec, lspec, lspec,
            pl.BlockSpec((D, D), lambda i: (0, 0)),
            pl.BlockSpec((D, DFF), lambda i: (0, 0)),
            pl.BlockSpec((1, DFF), lambda i: (0, 0)),
            pl.BlockSpec((DFF, D), lambda i: (0, 0)),
            pl.BlockSpec((1, D), lambda i: (0, 0)),
        ],
        out_specs=[
            pl.BlockSpec((TM, D), lambda i: (i, 0)),
            pl.BlockSpec((TM, D), lambda i: (i, 0)),
        ],
        out_shape=[
            jax.ShapeDtypeStruct((B * T, D), jnp.float32),
            jax.ShapeDtypeStruct((B * T, D), jnp.float32),
        ],
        interpret=interpret,
    )(x1.reshape(B * T, D), x2.reshape(B * T, D), o0, o1, l0, l1,
      W_o.astype(jnp.bfloat16), W_ff1.astype(jnp.bfloat16),
      b_ff1.reshape(1, DFF), W_ff2.astype(jnp.bfloat16),
      b_ff2.reshape(1, D))
    return y1.reshape(B, T, D), y2.reshape(B, T, D)


def kernel(x1, x2, W_qk, W_v, W_o, W_ff1, b_ff1, W_ff2, b_ff2, rotations,
           interpret=False):
    qkv = _proj(x2, W_qk, W_v, interpret=interpret)
    buckets = _buckets(qkv, rotations, interpret=interpret)
    ous, lus = [], []
    for r in range(R):
        sqkv, cb, dst = _sc_route(buckets[r], qkv)
        o, l = _attention(sqkv, cb, interpret=interpret)
        o_u, l_u = _sc_unsort(o, l, dst)
        ous.append(o_u)
        lus.append(l_u)
    return _tail(x1, x2, ous[0], ous[1], lus[0], lus[1],
                 W_o, W_ff1, b_ff1, W_ff2, b_ff2, interpret=interpret)


# speedup vs baseline: 1.1489x; 1.1489x over previous
import functools
import jax
import jax.numpy as jnp
import numpy as np
from jax.experimental import pallas as pl
from jax.experimental.pallas import tpu as pltpu
from jax.experimental.pallas import tpu_sc as plsc

B, T, D, H, DH, DFF = 2, 4096, 1024, 16, 64, 4096
R, C = 2, 64
NB = T // C
NC = T // C
G = 8
GR = G * C
NG = NC // G
RBH = R * B * H
BH = B * H


def _proj_body(x_ref, wc_ref, qkv_ref):
    qkv_ref[...] = jnp.dot(x_ref[...], wc_ref[...],
                           preferred_element_type=jnp.float32)


def _proj(x2, W_qk, W_v):
    Wc = jnp.concatenate([W_qk.reshape(D, H, 1, DH), W_v.reshape(D, H, 1, DH)],
                         axis=2).reshape(D, 2 * D)
    TM = 512
    grid = (B * T // TM,)
    return pl.pallas_call(
        _proj_body,
        grid=grid,
        in_specs=[
            pl.BlockSpec((TM, D), lambda i: (i, 0)),
            pl.BlockSpec((D, 2 * D), lambda i: (0, 0)),
        ],
        out_specs=pl.BlockSpec((TM, 2 * D), lambda i: (i, 0)),
        out_shape=jax.ShapeDtypeStruct((B * T, 2 * D), jnp.float32),
    )(x2.reshape(B * T, D), Wc)


def _bucket_body(qkr_ref, rot_ref, out_ref):
    rows = qkr_ref[:, :DH]
    rot = rot_ref[...]
    ro = jnp.dot(rows, rot, preferred_element_type=jnp.float32)
    cols = []
    half = NB // 2
    for r in range(R):
        rr = ro[:, r * half:(r + 1) * half]
        mp = jnp.max(rr, axis=-1)
        mn = jnp.max(-rr, axis=-1)
        ap = jnp.argmax(rr, axis=-1).astype(jnp.int32)
        an = jnp.argmax(-rr, axis=-1).astype(jnp.int32)
        b = jnp.where(mp >= mn, ap, half + an)
        cols.append(b[:, None])
    out_ref[...] = jnp.concatenate(cols, axis=1)


def _buckets(qkv, rotations):
    rot = rotations.transpose(1, 0, 2).reshape(DH, R * (NB // 2))
    TMH = 4096
    grid = (B * T * H // TMH,)
    out = pl.pallas_call(
        _bucket_body,
        grid=grid,
        in_specs=[
            pl.BlockSpec((TMH, 2 * DH), lambda i: (i, 0)),
            pl.BlockSpec((DH, R * (NB // 2)), lambda i: (0, 0)),
        ],
        out_specs=pl.BlockSpec((TMH, R), lambda i: (i, 0)),
        out_shape=jax.ShapeDtypeStruct((B * T * H, R), jnp.int32),
    )(qkv.reshape(B * T * H, 2 * DH), rot)
    return out.reshape(B, T, H, R).transpose(3, 0, 2, 1)


CH = 128
NCH = T // CH


def _sc_route_body(bkt_hbm, qkvrows_hbm,
                   sqkv_hbm, cb_hbm, dst_hbm,
                   bkt_v, rank_v, hist, off, ord_v, cb_v, dst_v, idx_v,
                   rowbuf, rowbuf2, sem, gsem, gsem2):
    ncores = 2
    wid = jax.lax.axis_index("s") * ncores + jax.lax.axis_index("c")
    lane = jax.lax.broadcasted_iota(jnp.int32, (16,), 0)
    bh = wid
    b = bh // H
    h = bh % H
    row_base = b * (T * H) + h
    pltpu.sync_copy(bkt_hbm.at[bh], bkt_v)
    for j in range(NB // 16):
        hist[pl.ds(j * 16, 16)] = jnp.zeros((16,), jnp.int32)

    def hist_step(i, _):
        b16 = bkt_v[pl.ds(i * 16, 16)]
        cnt, last = plsc.scan_count(b16)
        old = plsc.load_gather(hist, [b16])
        rank_v[pl.ds(i * 16, 16)] = old + cnt - 1
        plsc.addupdate_scatter(hist, [b16], cnt, mask=last)
        return 0

    jax.lax.fori_loop(0, T // 16, hist_step, 0)
    carry = jnp.int32(0)
    for j in range(NB // 16):
        h16 = hist[pl.ds(j * 16, 16)]
        inc = plsc.cumsum(h16) + carry
        off[pl.ds(j * 16, 16)] = inc - h16
        carry = carry + jnp.sum(h16)

    def perm_step(i, _):
        b16 = bkt_v[pl.ds(i * 16, 16)]
        r16 = rank_v[pl.ds(i * 16, 16)]
        d16 = plsc.load_gather(off, [b16]) + r16
        src = i * 16 + lane
        plsc.store_scatter(ord_v, [d16], src)
        plsc.store_scatter(cb_v, [d16], b16)
        dst_v[pl.ds(i * 16, 16)] = d16
        return 0

    jax.lax.fori_loop(0, T // 16, perm_step, 0)

    def idx_step(i, _):
        o16 = ord_v[pl.ds(i * 16, 16)]
        idx_v[pl.ds(i * 16, 16)] = o16 * H + row_base
        return 0

    jax.lax.fori_loop(0, T // 16, idx_step, 0)
    pltpu.sync_copy(cb_v, cb_hbm.at[bh])
    pltpu.sync_copy(dst_v, dst_hbm.at[bh])
    bufs = (rowbuf, rowbuf2)
    sems = (gsem, gsem2)
    cp = [None] * NCH
    for c in range(NCH):
        sl = pl.ds(c * CH, CH)
        cp[c] = pltpu.async_copy(qkvrows_hbm.at[idx_v.at[sl]],
                                 bufs[c % 2], sems[c % 2])
        if c >= 1:
            cp[c - 1].wait()
            pltpu.sync_copy(bufs[(c - 1) % 2],
                            sqkv_hbm.at[bh, pl.ds((c - 1) * CH, CH), :])
    cp[NCH - 1].wait()
    pltpu.sync_copy(bufs[(NCH - 1) % 2],
                    sqkv_hbm.at[bh, pl.ds((NCH - 1) * CH, CH), :])


def _sc_route(buckets_r, qkv):
    mesh = plsc.VectorSubcoreMesh(core_axis_name="c", subcore_axis_name="s")
    f = pl.kernel(
        _sc_route_body,
        compiler_params=pltpu.CompilerParams(needs_layout_passes=False),
        out_type=[
            jax.ShapeDtypeStruct((BH, T, 2 * DH), jnp.float32),
            jax.ShapeDtypeStruct((BH, T), jnp.int32),
            jax.ShapeDtypeStruct((BH, T), jnp.int32),
        ],
        mesh=mesh,
        scratch_types=[
            pltpu.VMEM((T,), jnp.int32),
            pltpu.VMEM((T,), jnp.int32),
            pltpu.VMEM((NB,), jnp.int32),
            pltpu.VMEM((NB,), jnp.int32),
            pltpu.VMEM((T,), jnp.int32),
            pltpu.VMEM((T,), jnp.int32),
            pltpu.VMEM((T,), jnp.int32),
            pltpu.VMEM((T,), jnp.int32),
            pltpu.VMEM((CH, 2 * DH), jnp.float32),
            pltpu.VMEM((CH, 2 * DH), jnp.float32),
            pltpu.SemaphoreType.DMA,
            pltpu.SemaphoreType.DMA,
            pltpu.SemaphoreType.DMA,
        ],
    )
    return f(buckets_r.reshape(BH, T), qkv.reshape(B * T * H, 2 * DH))


def _attn_body(sqkv_ref, bl_ref, o_ref, l_ref):
    scale = 1.0 / np.sqrt(DH)
    ione = jnp.ones((1, 1), jnp.float32)
    dn0 = (((0,), (0,)), ((), ()))
    qc = jax.lax.broadcasted_iota(jnp.int32, (GR, 1), 0) // C
    kcB = jax.lax.broadcasted_iota(jnp.int32, (1, GR), 1) // C
    allowB = (kcB == qc) | (kcB == qc - 1)
    allowP = qc == 0
    eye = (jax.lax.broadcasted_iota(jnp.int32, (GR, 1), 0)
           == jax.lax.broadcasted_iota(jnp.int32, (1, GR), 1))
    for g in range(NG):
        base = g * GR
        wstart = (base - C) % T
        qs = sqkv_ref[0, pl.ds(base, GR), :DH]
        kp = sqkv_ref[0, pl.ds(wstart, C), :DH]
        vb = sqkv_ref[0, pl.ds(base, GR), DH:]
        vp = sqkv_ref[0, pl.ds(wstart, C), DH:]
        bkb = bl_ref[0, :, pl.ds(base, GR)]
        bkp = bl_ref[0, :, pl.ds(wstart, C)]
        bq = jax.lax.dot_general(bkb.astype(jnp.float32), ione, dn0,
                                 preferred_element_type=jnp.float32)
        bq = bq.astype(jnp.int32)

        knb = qs / (jnp.sqrt(jnp.sum(qs * qs, axis=-1, keepdims=True)) + 1e-6)
        knp = kp / (jnp.sqrt(jnp.sum(kp * kp, axis=-1, keepdims=True)) + 1e-6)

        dn = (((1,), (1,)), ((), ()))
        qsb = qs.astype(jnp.bfloat16)
        dotsB = jax.lax.dot_general(qsb, knb.astype(jnp.bfloat16), dn,
                                    preferred_element_type=jnp.float32) * scale
        dotsP = jax.lax.dot_general(qsb, knp.astype(jnp.bfloat16), dn,
                                    preferred_element_type=jnp.float32) * scale

        maskB = (bq == bkb) & allowB
        dotsB = jnp.where(maskB, dotsB, -1e9)
        dotsB = jnp.where(eye, dotsB - 1e5, dotsB)

        maskP = (bq == bkp) & allowP
        dotsP = jnp.where(maskP, dotsP, -1e9)

        mB = jnp.max(dotsB, axis=-1, keepdims=True)
        mP = jnp.max(dotsP, axis=-1, keepdims=True)
        m = jnp.maximum(mB, mP)
        eB = jnp.exp(dotsB - m)
        eP = jnp.exp(dotsP - m)
        s = (jnp.sum(eB, axis=-1, keepdims=True)
             + jnp.sum(eP, axis=-1, keepdims=True))
        logits = m + jnp.log(s)
        o = (jnp.dot(eB.astype(jnp.bfloat16), vb.astype(jnp.bfloat16),
                     preferred_element_type=jnp.float32)
             + jnp.dot(eP.astype(jnp.bfloat16), vp.astype(jnp.bfloat16),
                       preferred_element_type=jnp.float32)) / s
        o_ref[0, pl.ds(base, GR), :DH] = o
        l_ref[0, pl.ds(base, GR), :] = logits


def _attention(sqkv, cb):
    grid = (BH,)
    o, l = pl.pallas_call(
        _attn_body,
        grid=grid,
        in_specs=[
            pl.BlockSpec((1, T, 2 * DH), lambda i: (i, 0, 0)),
            pl.BlockSpec((1, 1, T), lambda i: (i, 0, 0)),
        ],
        out_specs=[
            pl.BlockSpec((1, T, 2 * DH), lambda i: (i, 0, 0)),
            pl.BlockSpec((1, T, 1), lambda i: (i, 0, 0)),
        ],
        out_shape=[
            jax.ShapeDtypeStruct((BH, T, 2 * DH), jnp.float32),
            jax.ShapeDtypeStruct((BH, T, 1), jnp.float32),
        ],
    )(sqkv, cb.reshape(BH, 1, T))
    return o, l


def _sc_unsort_body(orows_hbm, l_hbm, dst_hbm,
                    ou_hbm, lu_hbm,
                    dst_v, idx_v, l_v, lu_v, rowbuf, rowbuf2,
                    sem, gsem, gsem2):
    ncores = 2
    wid = jax.lax.axis_index("s") * ncores + jax.lax.axis_index("c")
    bh = wid
    b = bh // H
    h = bh % H
    pltpu.sync_copy(dst_hbm.at[bh], dst_v)
    pltpu.sync_copy(l_hbm.at[bh], l_v)

    def idx_step(i, _):
        d16 = dst_v[pl.ds(i * 16, 16)]
        idx_v[pl.ds(i * 16, 16)] = d16 + bh * T
        lu_v[pl.ds(i * 16, 16)] = plsc.load_gather(l_v, [d16])
        return 0

    jax.lax.fori_loop(0, T // 16, idx_step, 0)
    pltpu.sync_copy(lu_v, lu_hbm.at[b, h, :])
    bufs = (rowbuf, rowbuf2)
    sems = (gsem, gsem2)
    cp = [None] * NCH
    for c in range(NCH):
        sl = pl.ds(c * CH, CH)
        cp[c] = pltpu.async_copy(orows_hbm.at[idx_v.at[sl]],
                                 bufs[c % 2], sems[c % 2])
        if c >= 1:
            cp[c - 1].wait()
            pltpu.sync_copy(bufs[(c - 1) % 2],
                            ou_hbm.at[b, h, pl.ds((c - 1) * CH, CH), :])
    cp[NCH - 1].wait()
    pltpu.sync_copy(bufs[(NCH - 1) % 2],
                    ou_hbm.at[b, h, pl.ds((NCH - 1) * CH, CH), :])


def _sc_unsort(o, l, dst):
    mesh = plsc.VectorSubcoreMesh(core_axis_name="c", subcore_axis_name="s")
    f = pl.kernel(
        _sc_unsort_body,
        compiler_params=pltpu.CompilerParams(needs_layout_passes=False),
        out_type=[
            jax.ShapeDtypeStruct((B, H, T, 2 * DH), jnp.float32),
            jax.ShapeDtypeStruct((B, H, T), jnp.float32),
        ],
        mesh=mesh,
        scratch_types=[
            pltpu.VMEM((T,), jnp.int32),
            pltpu.VMEM((T,), jnp.int32),
            pltpu.VMEM((T,), jnp.float32),
            pltpu.VMEM((T,), jnp.float32),
            pltpu.VMEM((CH, 2 * DH), jnp.float32),
            pltpu.VMEM((CH, 2 * DH), jnp.float32),
            pltpu.SemaphoreType.DMA,
            pltpu.SemaphoreType.DMA,
            pltpu.SemaphoreType.DMA,
        ],
    )
    o_u, l_u = f(o.reshape(BH * T, 2 * DH), l.reshape(BH, T), dst)
    return o_u, l_u


def _tail_body(x1_ref, x2_ref, o0_ref, o1_ref, l0_ref, l1_ref,
               wo_ref, w1_ref, b1_ref, w2_ref, b2_ref, y1_ref, y2_ref):
    l0 = l0_ref[0, :, :]
    l1 = l1_ref[0, :, :]
    m = jnp.maximum(l0, l1)
    e0 = jnp.exp(l0 - m)
    e1 = jnp.exp(l1 - m)
    den = e0 + e1
    w0 = e0 / den
    w1 = e1 / den
    ex = (jax.lax.broadcasted_iota(jnp.int32, (H, D), 1) // DH
          == jax.lax.broadcasted_iota(jnp.int32, (H, D), 0)).astype(jnp.float32)
    dn = (((0,), (0,)), ((), ()))
    w0e = jax.lax.dot_general(w0, ex, dn,
                              preferred_element_type=jnp.float32)
    w1e = jax.lax.dot_general(w1, ex, dn,
                              preferred_element_type=jnp.float32)
    parts = []
    for h in range(H):
        sl = slice(h * DH, (h + 1) * DH)
        parts.append(w0e[:, sl] * o0_ref[0, h, :, :DH]
                     + w1e[:, sl] * o1_ref[0, h, :, :DH])
    attn = jnp.concatenate(parts, axis=1)
    y1 = x1_ref[...] + jnp.dot(attn.astype(jnp.bfloat16), wo_ref[...],
                               preferred_element_type=jnp.float32)
    y1_ref[...] = y1
    hpre = jnp.dot(y1.astype(jnp.bfloat16), w1_ref[...],
                   preferred_element_type=jnp.float32)
    hpre = hpre + b1_ref[...]
    hh = jax.nn.gelu(hpre)
    y2 = x2_ref[...] + jnp.dot(hh.astype(jnp.bfloat16), w2_ref[...],
                               preferred_element_type=jnp.float32) + b2_ref[...]
    y2_ref[...] = y2


def _tail(x1, x2, o0, o1, l0, l1, W_o, W_ff1, b_ff1, W_ff2, b_ff2):
    TM = 256
    NT = T // TM
    grid = (B * T // TM,)
    ospec = pl.BlockSpec((1, H, TM, 2 * DH), lambda i: (i // NT, 0, i % NT, 0))
    lspec = pl.BlockSpec((1, H, TM), lambda i: (i // NT, 0, i % NT))
    y1, y2 = pl.pallas_call(
        _tail_body,
        grid=grid,
        in_specs=[
            pl.BlockSpec((TM, D), lambda i: (i, 0)),
            pl.BlockSpec((TM, D), lambda i: (i, 0)),
            ospec, ospec, lspec, lspec,
            pl.BlockSpec((D, D), lambda i: (0, 0)),
            pl.BlockSpec((D, DFF), lambda i: (0, 0)),
            pl.BlockSpec((1, DFF), lambda i: (0, 0)),
            pl.BlockSpec((DFF, D), lambda i: (0, 0)),
            pl.BlockSpec((1, D), lambda i: (0, 0)),
        ],
        out_specs=[
            pl.BlockSpec((TM, D), lambda i: (i, 0)),
            pl.BlockSpec((TM, D), lambda i: (i, 0)),
        ],
        out_shape=[
            jax.ShapeDtypeStruct((B * T, D), jnp.float32),
            jax.ShapeDtypeStruct((B * T, D), jnp.float32),
        ],
    )(x1.reshape(B * T, D), x2.reshape(B * T, D), o0, o1, l0, l1,
      W_o.astype(jnp.bfloat16), W_ff1.astype(jnp.bfloat16),
      b_ff1.reshape(1, DFF), W_ff2.astype(jnp.bfloat16),
      b_ff2.reshape(1, D))
    return y1.reshape(B, T, D), y2.reshape(B, T, D)


def kernel(x1, x2, W_qk, W_v, W_o, W_ff1, b_ff1, W_ff2, b_ff2, rotations):
    qkv = _proj(x2, W_qk, W_v)
    buckets = _buckets(qkv, rotations)
    ous, lus = [], []
    for r in range(R):
        sqkv, cb, dst = _sc_route(buckets[r], qkv)
        o, l = _attention(sqkv, cb)
        o_u, l_u = _sc_unsort(o, l, dst)
        ous.append(o_u)
        lus.append(l_u)
    return _tail(x1, x2, ous[0], ous[1], lus[0], lus[1],
                 W_o, W_ff1, b_ff1, W_ff2, b_ff2)


# speedup vs baseline: 1.1504x; 1.0013x over previous
import functools
import jax
import jax.numpy as jnp
import numpy as np
from jax.experimental import pallas as pl
from jax.experimental.pallas import tpu as pltpu
from jax.experimental.pallas import tpu_sc as plsc

B, T, D, H, DH, DFF = 2, 4096, 1024, 16, 64, 4096
R, C = 2, 64
NB = T // C
NC = T // C
G = 8
GR = G * C
NG = NC // G
RBH = R * B * H
BH = B * H


def _proj_body(x_ref, wc_ref, qkv_ref):
    qkv_ref[...] = jnp.dot(x_ref[...], wc_ref[...],
                           preferred_element_type=jnp.float32)


def _proj(x2, W_qk, W_v):
    Wc = jnp.concatenate([W_qk.reshape(D, H, 1, DH), W_v.reshape(D, H, 1, DH)],
                         axis=2).reshape(D, 2 * D)
    TM = 512
    grid = (B * T // TM,)
    return pl.pallas_call(
        _proj_body,
        grid=grid,
        in_specs=[
            pl.BlockSpec((TM, D), lambda i: (i, 0)),
            pl.BlockSpec((D, 2 * D), lambda i: (0, 0)),
        ],
        out_specs=pl.BlockSpec((TM, 2 * D), lambda i: (i, 0)),
        out_shape=jax.ShapeDtypeStruct((B * T, 2 * D), jnp.float32),
    )(x2.reshape(B * T, D), Wc)


def _bucket_body(qkr_ref, rot_ref, out_ref):
    rows = qkr_ref[:, :DH]
    rot = rot_ref[...]
    ro = jnp.dot(rows, rot, preferred_element_type=jnp.float32)
    cols = []
    half = NB // 2
    for r in range(R):
        rr = ro[:, r * half:(r + 1) * half]
        mp = jnp.max(rr, axis=-1)
        mn = jnp.max(-rr, axis=-1)
        ap = jnp.argmax(rr, axis=-1).astype(jnp.int32)
        an = jnp.argmax(-rr, axis=-1).astype(jnp.int32)
        b = jnp.where(mp >= mn, ap, half + an)
        cols.append(b[:, None])
    out_ref[...] = jnp.concatenate(cols, axis=1)


def _buckets(qkv, rotations):
    rot = rotations.transpose(1, 0, 2).reshape(DH, R * (NB // 2))
    TMH = 4096
    grid = (B * T * H // TMH,)
    out = pl.pallas_call(
        _bucket_body,
        grid=grid,
        in_specs=[
            pl.BlockSpec((TMH, 2 * DH), lambda i: (i, 0)),
            pl.BlockSpec((DH, R * (NB // 2)), lambda i: (0, 0)),
        ],
        out_specs=pl.BlockSpec((TMH, R), lambda i: (i, 0)),
        out_shape=jax.ShapeDtypeStruct((B * T * H, R), jnp.int32),
    )(qkv.reshape(B * T * H, 2 * DH), rot)
    return out.reshape(B, T, H, R).transpose(3, 0, 2, 1)


CH = 128
NCH = T // CH


def _sc_route_body(bkt_hbm, qkvrows_hbm,
                   sqkv_hbm, cb_hbm, dst_hbm,
                   bkt_v, rank_v, hist, off, cb_v, dst_v, idx_v,
                   rowbuf, rowbuf2, sem, gsem, gsem2):
    ncores = 2
    wid = jax.lax.axis_index("s") * ncores + jax.lax.axis_index("c")
    lane = jax.lax.broadcasted_iota(jnp.int32, (16,), 0)
    bh = wid
    b = bh // H
    h = bh % H
    row_base = b * (T * H) + h
    pltpu.sync_copy(bkt_hbm.at[bh], bkt_v)
    for j in range(NB // 16):
        hist[pl.ds(j * 16, 16)] = jnp.zeros((16,), jnp.int32)

    def hist_step(i, _):
        b16 = bkt_v[pl.ds(i * 16, 16)]
        cnt, last = plsc.scan_count(b16)
        old = plsc.load_gather(hist, [b16])
        rank_v[pl.ds(i * 16, 16)] = old + cnt - 1
        plsc.addupdate_scatter(hist, [b16], cnt, mask=last)
        return 0

    jax.lax.fori_loop(0, T // 16, hist_step, 0)
    carry = jnp.int32(0)
    for j in range(NB // 16):
        h16 = hist[pl.ds(j * 16, 16)]
        inc = plsc.cumsum(h16) + carry
        off[pl.ds(j * 16, 16)] = inc - h16
        carry = carry + jnp.sum(h16)

    def perm_step(i, _):
        b16 = bkt_v[pl.ds(i * 16, 16)]
        r16 = rank_v[pl.ds(i * 16, 16)]
        d16 = plsc.load_gather(off, [b16]) + r16
        src = i * 16 + lane
        plsc.store_scatter(idx_v, [d16], src * H + row_base)
        plsc.store_scatter(cb_v, [d16], b16)
        dst_v[pl.ds(i * 16, 16)] = d16
        return 0

    jax.lax.fori_loop(0, T // 16, perm_step, 0)
    pltpu.sync_copy(cb_v, cb_hbm.at[bh])
    pltpu.sync_copy(dst_v, dst_hbm.at[bh])
    bufs = (rowbuf, rowbuf2)
    sems = (gsem, gsem2)
    cp = [None] * NCH
    for c in range(NCH):
        sl = pl.ds(c * CH, CH)
        cp[c] = pltpu.async_copy(qkvrows_hbm.at[idx_v.at[sl]],
                                 bufs[c % 2], sems[c % 2])
        if c >= 1:
            cp[c - 1].wait()
            pltpu.sync_copy(bufs[(c - 1) % 2],
                            sqkv_hbm.at[bh, pl.ds((c - 1) * CH, CH), :])
    cp[NCH - 1].wait()
    pltpu.sync_copy(bufs[(NCH - 1) % 2],
                    sqkv_hbm.at[bh, pl.ds((NCH - 1) * CH, CH), :])


def _sc_route(buckets_r, qkv):
    mesh = plsc.VectorSubcoreMesh(core_axis_name="c", subcore_axis_name="s")
    f = pl.kernel(
        _sc_route_body,
        compiler_params=pltpu.CompilerParams(needs_layout_passes=False),
        out_type=[
            jax.ShapeDtypeStruct((BH, T, 2 * DH), jnp.float32),
            jax.ShapeDtypeStruct((BH, T), jnp.int32),
            jax.ShapeDtypeStruct((BH, T), jnp.int32),
        ],
        mesh=mesh,
        scratch_types=[
            pltpu.VMEM((T,), jnp.int32),
            pltpu.VMEM((T,), jnp.int32),
            pltpu.VMEM((NB,), jnp.int32),
            pltpu.VMEM((NB,), jnp.int32),
            pltpu.VMEM((T,), jnp.int32),
            pltpu.VMEM((T,), jnp.int32),
            pltpu.VMEM((T,), jnp.int32),
            pltpu.VMEM((CH, 2 * DH), jnp.float32),
            pltpu.VMEM((CH, 2 * DH), jnp.float32),
            pltpu.SemaphoreType.DMA,
            pltpu.SemaphoreType.DMA,
            pltpu.SemaphoreType.DMA,
        ],
    )
    return f(buckets_r.reshape(BH, T), qkv.reshape(B * T * H, 2 * DH))


def _attn_body(sqkv_ref, bl_ref, o_ref, l_ref):
    scale = 1.0 / np.sqrt(DH)
    ione = jnp.ones((1, 1), jnp.float32)
    dn0 = (((0,), (0,)), ((), ()))
    qc = jax.lax.broadcasted_iota(jnp.int32, (GR, 1), 0) // C
    kcB = jax.lax.broadcasted_iota(jnp.int32, (1, GR), 1) // C
    allowB = (kcB == qc) | (kcB == qc - 1)
    allowP = qc == 0
    eye = (jax.lax.broadcasted_iota(jnp.int32, (GR, 1), 0)
           == jax.lax.broadcasted_iota(jnp.int32, (1, GR), 1))
    for g in range(NG):
        base = g * GR
        wstart = (base - C) % T
        qs = sqkv_ref[0, pl.ds(base, GR), :DH]
        kp = sqkv_ref[0, pl.ds(wstart, C), :DH]
        vb = sqkv_ref[0, pl.ds(base, GR), DH:]
        vp = sqkv_ref[0, pl.ds(wstart, C), DH:]
        bkb = bl_ref[0, :, pl.ds(base, GR)]
        bkp = bl_ref[0, :, pl.ds(wstart, C)]
        bq = jax.lax.dot_general(bkb.astype(jnp.float32), ione, dn0,
                                 preferred_element_type=jnp.float32)
        bq = bq.astype(jnp.int32)

        knb = qs / (jnp.sqrt(jnp.sum(qs * qs, axis=-1, keepdims=True)) + 1e-6)
        knp = kp / (jnp.sqrt(jnp.sum(kp * kp, axis=-1, keepdims=True)) + 1e-6)

        dn = (((1,), (1,)), ((), ()))
        qsb = qs.astype(jnp.bfloat16)
        dotsB = jax.lax.dot_general(qsb, knb.astype(jnp.bfloat16), dn,
                                    preferred_element_type=jnp.float32) * scale
        dotsP = jax.lax.dot_general(qsb, knp.astype(jnp.bfloat16), dn,
                                    preferred_element_type=jnp.float32) * scale

        maskB = (bq == bkb) & allowB
        dotsB = jnp.where(maskB, dotsB, -1e9)
        dotsB = jnp.where(eye, dotsB - 1e5, dotsB)

        maskP = (bq == bkp) & allowP
        dotsP = jnp.where(maskP, dotsP, -1e9)

        mB = jnp.max(dotsB, axis=-1, keepdims=True)
        mP = jnp.max(dotsP, axis=-1, keepdims=True)
        m = jnp.maximum(mB, mP)
        eB = jnp.exp(dotsB - m)
        eP = jnp.exp(dotsP - m)
        s = (jnp.sum(eB, axis=-1, keepdims=True)
             + jnp.sum(eP, axis=-1, keepdims=True))
        logits = m + jnp.log(s)
        o = (jnp.dot(eB.astype(jnp.bfloat16), vb.astype(jnp.bfloat16),
                     preferred_element_type=jnp.float32)
             + jnp.dot(eP.astype(jnp.bfloat16), vp.astype(jnp.bfloat16),
                       preferred_element_type=jnp.float32)) / s
        o_ref[0, pl.ds(base, GR), :DH] = o
        l_ref[0, pl.ds(base, GR), :] = logits


def _attention(sqkv, cb):
    grid = (BH,)
    o, l = pl.pallas_call(
        _attn_body,
        grid=grid,
        in_specs=[
            pl.BlockSpec((1, T, 2 * DH), lambda i: (i, 0, 0)),
            pl.BlockSpec((1, 1, T), lambda i: (i, 0, 0)),
        ],
        out_specs=[
            pl.BlockSpec((1, T, 2 * DH), lambda i: (i, 0, 0)),
            pl.BlockSpec((1, T, 1), lambda i: (i, 0, 0)),
        ],
        out_shape=[
            jax.ShapeDtypeStruct((BH, T, 2 * DH), jnp.float32),
            jax.ShapeDtypeStruct((BH, T, 1), jnp.float32),
        ],
    )(sqkv, cb.reshape(BH, 1, T))
    return o, l


def _sc_unsort_body(orows_hbm, l_hbm, dst_hbm,
                    ou_hbm, lu_hbm,
                    dst_v, idx_v, l_v, lu_v, rowbuf, rowbuf2,
                    sem, gsem, gsem2):
    ncores = 2
    wid = jax.lax.axis_index("s") * ncores + jax.lax.axis_index("c")
    bh = wid
    b = bh // H
    h = bh % H
    pltpu.sync_copy(dst_hbm.at[bh], dst_v)
    pltpu.sync_copy(l_hbm.at[bh], l_v)

    def idx_step(i, _):
        d16 = dst_v[pl.ds(i * 16, 16)]
        idx_v[pl.ds(i * 16, 16)] = d16 + bh * T
        lu_v[pl.ds(i * 16, 16)] = plsc.load_gather(l_v, [d16])
        return 0

    jax.lax.fori_loop(0, T // 16, idx_step, 0)
    pltpu.sync_copy(lu_v, lu_hbm.at[b, h, :])
    bufs = (rowbuf, rowbuf2)
    sems = (gsem, gsem2)
    cp = [None] * NCH
    for c in range(NCH):
        sl = pl.ds(c * CH, CH)
        cp[c] = pltpu.async_copy(orows_hbm.at[idx_v.at[sl]],
                                 bufs[c % 2], sems[c % 2])
        if c >= 1:
            cp[c - 1].wait()
            pltpu.sync_copy(bufs[(c - 1) % 2],
                            ou_hbm.at[b, h, pl.ds((c - 1) * CH, CH), :])
    cp[NCH - 1].wait()
    pltpu.sync_copy(bufs[(NCH - 1) % 2],
                    ou_hbm.at[b, h, pl.ds((NCH - 1) * CH, CH), :])


def _sc_unsort(o, l, dst):
    mesh = plsc.VectorSubcoreMesh(core_axis_name="c", subcore_axis_name="s")
    f = pl.kernel(
        _sc_unsort_body,
        compiler_params=pltpu.CompilerParams(needs_layout_passes=False),
        out_type=[
            jax.ShapeDtypeStruct((B, H, T, 2 * DH), jnp.float32),
            jax.ShapeDtypeStruct((B, H, T), jnp.float32),
        ],
        mesh=mesh,
        scratch_types=[
            pltpu.VMEM((T,), jnp.int32),
            pltpu.VMEM((T,), jnp.int32),
            pltpu.VMEM((T,), jnp.float32),
            pltpu.VMEM((T,), jnp.float32),
            pltpu.VMEM((CH, 2 * DH), jnp.float32),
            pltpu.VMEM((CH, 2 * DH), jnp.float32),
            pltpu.SemaphoreType.DMA,
            pltpu.SemaphoreType.DMA,
            pltpu.SemaphoreType.DMA,
        ],
    )
    o_u, l_u = f(o.reshape(BH * T, 2 * DH), l.reshape(BH, T), dst)
    return o_u, l_u


def _tail_body(x1_ref, x2_ref, o0_ref, o1_ref, l0_ref, l1_ref,
               wo_ref, w1_ref, b1_ref, w2_ref, b2_ref, y1_ref, y2_ref):
    l0 = l0_ref[0, :, :]
    l1 = l1_ref[0, :, :]
    m = jnp.maximum(l0, l1)
    e0 = jnp.exp(l0 - m)
    e1 = jnp.exp(l1 - m)
    den = e0 + e1
    w0 = e0 / den
    w1 = e1 / den
    ex = (jax.lax.broadcasted_iota(jnp.int32, (H, D), 1) // DH
          == jax.lax.broadcasted_iota(jnp.int32, (H, D), 0)).astype(jnp.float32)
    dn = (((0,), (0,)), ((), ()))
    w0e = jax.lax.dot_general(w0, ex, dn,
                              preferred_element_type=jnp.float32)
    w1e = jax.lax.dot_general(w1, ex, dn,
                              preferred_element_type=jnp.float32)
    parts = []
    for h in range(H):
        sl = slice(h * DH, (h + 1) * DH)
        parts.append(w0e[:, sl] * o0_ref[0, h, :, :DH]
                     + w1e[:, sl] * o1_ref[0, h, :, :DH])
    attn = jnp.concatenate(parts, axis=1)
    y1 = x1_ref[...] + jnp.dot(attn.astype(jnp.bfloat16), wo_ref[...],
                               preferred_element_type=jnp.float32)
    y1_ref[...] = y1
    hpre = jnp.dot(y1.astype(jnp.bfloat16), w1_ref[...],
                   preferred_element_type=jnp.float32)
    hpre = hpre + b1_ref[...]
    hh = jax.nn.gelu(hpre)
    y2 = x2_ref[...] + jnp.dot(hh.astype(jnp.bfloat16), w2_ref[...],
                               preferred_element_type=jnp.float32) + b2_ref[...]
    y2_ref[...] = y2


def _tail(x1, x2, o0, o1, l0, l1, W_o, W_ff1, b_ff1, W_ff2, b_ff2):
    TM = 256
    NT = T // TM
    grid = (B * T // TM,)
    ospec = pl.BlockSpec((1, H, TM, 2 * DH), lambda i: (i // NT, 0, i % NT, 0))
    lspec = pl.BlockSpec((1, H, TM), lambda i: (i // NT, 0, i % NT))
    y1, y2 = pl.pallas_call(
        _tail_body,
        grid=grid,
        in_specs=[
            pl.BlockSpec((TM, D), lambda i: (i, 0)),
            pl.BlockSpec((TM, D), lambda i: (i, 0)),
            ospec, ospec, lspec, lspec,
            pl.BlockSpec((D, D), lambda i: (0, 0)),
            pl.BlockSpec((D, DFF), lambda i: (0, 0)),
            pl.BlockSpec((1, DFF), lambda i: (0, 0)),
            pl.BlockSpec((DFF, D), lambda i: (0, 0)),
            pl.BlockSpec((1, D), lambda i: (0, 0)),
        ],
        out_specs=[
            pl.BlockSpec((TM, D), lambda i: (i, 0)),
            pl.BlockSpec((TM, D), lambda i: (i, 0)),
        ],
        out_shape=[
            jax.ShapeDtypeStruct((B * T, D), jnp.float32),
            jax.ShapeDtypeStruct((B * T, D), jnp.float32),
        ],
    )(x1.reshape(B * T, D), x2.reshape(B * T, D), o0, o1, l0, l1,
      W_o.astype(jnp.bfloat16), W_ff1.astype(jnp.bfloat16),
      b_ff1.reshape(1, DFF), W_ff2.astype(jnp.bfloat16),
      b_ff2.reshape(1, D))
    return y1.reshape(B, T, D), y2.reshape(B, T, D)


def kernel(x1, x2, W_qk, W_v, W_o, W_ff1, b_ff1, W_ff2, b_ff2, rotations):
    qkv = _proj(x2, W_qk, W_v)
    buckets = _buckets(qkv, rotations)
    ous, lus = [], []
    for r in range(R):
        sqkv, cb, dst = _sc_route(buckets[r], qkv)
        o, l = _attention(sqkv, cb)
        o_u, l_u = _sc_unsort(o, l, dst)
        ous.append(o_u)
        lus.append(l_u)
    return _tail(x1, x2, ous[0], ous[1], lus[0], lus[1],
                 W_o, W_ff1, b_ff1, W_ff2, b_ff2)
